# baseline (speedup 1.0000x reference)
import functools

import jax
import jax.numpy as jnp
from jax import lax
from jax.experimental import pallas as pl
from jax.experimental.pallas import tpu as pltpu
from jax.experimental.pallas import tpu_sc as plsc

N = 10000
E = 160000
F = 256
K = 5000

PN = 10240


def _mm_kernel(a_ref, b_ref, o_ref):
    o_ref[...] = jnp.dot(a_ref[...], b_ref[...], preferred_element_type=jnp.float32)


def _mm(a, b, bm):
    m, k = a.shape
    _, n = b.shape
    return pl.pallas_call(
        _mm_kernel,
        grid=(m // bm,),
        in_specs=[pl.BlockSpec((bm, k), lambda i: (i, 0)),
                  pl.BlockSpec((k, n), lambda i: (0, 0))],
        out_specs=pl.BlockSpec((bm, n), lambda i: (i, 0)),
        out_shape=jax.ShapeDtypeStruct((m, n), jnp.float32),
    )(a, b)


def _rank_kernel(si_ref, sall_ref, rank_ref):
    ib = pl.program_id(0)
    si = si_ref[...]
    gi = ib * 1024 + lax.broadcasted_iota(jnp.int32, (1024, 1), 0)
    acc = jnp.zeros((1024, 128), jnp.int32)
    for jc in range(PN // 128):
        sj = sall_ref[jc:jc + 1, :]
        gj = jc * 128 + lax.broadcasted_iota(jnp.int32, (1, 128), 1)
        gt = (sj > si)
        eq = (sj == si) & (gj < gi)
        acc = acc + (gt | eq).astype(jnp.int32)
    rank_ref[...] = jnp.sum(acc, axis=1, keepdims=True)


def _rank(scores_pad):
    s_col = scores_pad.reshape(PN, 1)
    s2d = scores_pad.reshape(PN // 128, 128)
    return pl.pallas_call(
        _rank_kernel,
        grid=(PN // 1024,),
        in_specs=[pl.BlockSpec((1024, 1), lambda i: (i, 0)),
                  pl.BlockSpec((PN // 128, 128), lambda i: (0, 0))],
        out_specs=pl.BlockSpec((1024, 1), lambda i: (i, 0)),
        out_shape=jax.ShapeDtypeStruct((PN, 1), jnp.int32),
    )(s_col, s2d).reshape(PN)[:N]


def _shard_bounds(u):
    sh = -(-(u // 16) // 16) * 16
    bounds = []
    p = 0
    t = 16
    while t > 2 and p + sh < u:
        bounds.append(p + sh)
        p += sh
        t -= 1
    rem = u - p
    half = -(-(rem + 1) // 2 // 16) * 16
    if rem > half:
        bounds.append(p + half)
    return bounds


def _exact_segsum(msg, dd, u, n_out, width):
    return jnp.zeros((n_out, width), jnp.float32).at[dd].add(msg)


_R = 160
_NRANGE = 64
_NPAD = _R * _NRANGE
_CH = 1280
_FL = 128


def _make_segsum(u, has_loop):
    bounds = _shard_bounds(u)
    nblk = E // _CH
    mesh = plsc.VectorSubcoreMesh(core_axis_name="c", subcore_axis_name="s")

    def vscalar(ref, idx):
        return ref[pl.ds(idx, 16)][0]

    @functools.partial(
        pl.kernel, mesh=mesh,
        out_type=jax.ShapeDtypeStruct((_NPAD, F), jnp.float32),
        scratch_types=[
            pltpu.VMEM((_R, F), jnp.float32),
            pltpu.VMEM((_R, F), jnp.float32),
            pltpu.VMEM((_FL, F), jnp.float32),
            pltpu.VMEM((_CH + 16,), jnp.int32),
            pltpu.VMEM((_CH + 16,), jnp.int32),
            pltpu.VMEM((_CH + 16,), jnp.float32),
            pltpu.VMEM((_FL + 32,), jnp.int32),
            pltpu.VMEM((E // 16 + 16,), jnp.int32),
            pltpu.VMEM((_R + 16,), jnp.int32),
            pltpu.VMEM((_R + 16,), jnp.float32),
            pltpu.SMEM((_FL + 16,), jnp.int32),
            pltpu.SMEM((_FL + 16,), jnp.float32),
            pltpu.SMEM((1,), jnp.int32),
            pltpu.SMEM((_R,), jnp.int32),
            pltpu.SMEM((_R,), jnp.float32),
            pltpu.SMEM((_R,), jnp.int32),
            pltpu.SMEM((_R,), jnp.int32),
            pltpu.SemaphoreType.DMA,
        ],
    )
    def seg_kernel(g_hbm, src_hbm, dst_hbm, sc_hbm, hit_hbm, off_hbm,
                   lsc_hbm, out_hbm,
                   t_v, p_v, rows_v, dblk_v, sblk_v, scblk_v,
                   gidx_v, hit_v, offst_v, lscst_v,
                   pd_s, pc_s, np_s, off_s, lsc_s, cnt_s, lastp_s, sem):
        wid = lax.axis_index("s") * 2 + lax.axis_index("c")
        zf16 = jnp.zeros((16,), jnp.float32)

        def piece_of(pos):
            pid = jnp.int32(0)
            for b in bounds:
                pid = pid + (pos >= b).astype(jnp.int32)
            return pid

        def accum(r, ld, scv):
            pos = off_s[ld] + cnt_s[ld]
            cnt_s[ld] = cnt_s[ld] + 1
            piece = piece_of(pos)
            lp = lastp_s[ld]
            lastp_s[ld] = piece

            @pl.when(piece > lp)
            def _():
                for cc in range(F // 16):
                    sl = pl.ds(cc * 16, 16)
                    t_v[ld, sl] = t_v[ld, sl] + p_v[ld, sl]
                    p_v[ld, sl] = zf16

            for cc in range(F // 16):
                sl = pl.ds(cc * 16, 16)
                p_v[ld, sl] = p_v[ld, sl] + rows_v[r, sl] * scv

        def gather_pending():
            pltpu.async_copy(g_hbm.at[gidx_v.at[pl.ds(0, _FL)]], rows_v,
                             sem).wait()

        for rng_i in range(2):
            rid = wid * 2 + rng_i
            lo = rid * _R

            pltpu.sync_copy(hit_hbm.at[pl.ds(rid * (E // 16), E // 16)],
                            hit_v.at[pl.ds(0, E // 16)])
            pltpu.sync_copy(off_hbm.at[pl.ds(lo, _R)],
                            offst_v.at[pl.ds(0, _R)])
            if has_loop:
                pltpu.sync_copy(lsc_hbm.at[pl.ds(lo, _R)],
                                lscst_v.at[pl.ds(0, _R)])

            def ibody(r, _):
                off_s[r] = vscalar(offst_v, r)
                if has_loop:
                    lsc_s[r] = vscalar(lscst_v, r)
                cnt_s[r] = 0
                lastp_s[r] = -1
                return 0
            lax.fori_loop(0, _R, ibody, 0)
            np_s[0] = 0

            for cc in range(F // 16):
                sl = pl.ds(cc * 16, 16)

                def zbody(r, _, sl=sl):
                    t_v[r, sl] = zf16
                    p_v[r, sl] = zf16
                    return 0
                lax.fori_loop(0, _R, zbody, 0)

            def blk_body(blk, _):
                eb = blk * _CH
                pltpu.sync_copy(dst_hbm.at[pl.ds(eb, _CH)],
                                dblk_v.at[pl.ds(0, _CH)])
                pltpu.sync_copy(src_hbm.at[pl.ds(eb, _CH)],
                                sblk_v.at[pl.ds(0, _CH)])
                pltpu.sync_copy(sc_hbm.at[pl.ds(eb, _CH)],
                                scblk_v.at[pl.ds(0, _CH)])

                def ch_body(ci, _):
                    base = ci * 16
                    c = vscalar(hit_v, blk * (_CH // 16) + ci)

                    @pl.when(c > 0)
                    def _():
                        def sift(r, _):
                            ld = vscalar(dblk_v, base + r) - lo

                            @pl.when((ld >= 0) & (ld < _R))
                            def _():
                                np0 = np_s[0]
                                pd_s[np0] = ld
                                pc_s[np0] = vscalar(scblk_v, base + r)
                                gidx_v[pl.ds(np0, 16)] = jnp.full(
                                    (16,), vscalar(sblk_v, base + r),
                                    jnp.int32)
                                np_s[0] = np0 + 1
                            return 0
                        lax.fori_loop(0, 16, sift, 0)

                    @pl.when(np_s[0] >= _FL)
                    def _():
                        gather_pending()

                        def fbody(r, _):
                            accum(r, pd_s[r], pc_s[r])
                            return 0
                        lax.fori_loop(0, _FL, fbody, 0)
                        rem = np_s[0] - _FL

                        def mbody(q, _):
                            @pl.when(q < rem)
                            def _():
                                pd_s[q] = pd_s[_FL + q]
                                pc_s[q] = pc_s[_FL + q]
                                gidx_v[pl.ds(q, 16)] = jnp.full(
                                    (16,), vscalar(gidx_v, _FL + q),
                                    jnp.int32)
                            return 0
                        lax.fori_loop(0, 16, mbody, 0)
                        np_s[0] = rem
                    return 0
                lax.fori_loop(0, _CH // 16, ch_body, 0)
                return 0
            lax.fori_loop(0, nblk, blk_body, 0)

            nleft = np_s[0]

            @pl.when(nleft > 0)
            def _():
                def padb(r, _):
                    @pl.when(r >= nleft)
                    def _():
                        gidx_v[pl.ds(r, 16)] = jnp.zeros((16,), jnp.int32)
                    return 0
                lax.fori_loop(0, _FL, padb, 0)
                gather_pending()

                def fbody(r, _):
                    @pl.when(r < nleft)
                    def _():
                        accum(r, pd_s[r], pc_s[r])
                    return 0
                lax.fori_loop(0, _FL, fbody, 0)

            if has_loop:
                def lb_body(b, _):
                    rb = b * 16
                    pltpu.async_copy(g_hbm.at[pl.ds(lo + rb, 16)],
                                     rows_v.at[pl.ds(0, 16)], sem).wait()

                    def rbody(r, _):
                        accum(r, rb + r, lsc_s[rb + r])
                        return 0
                    lax.fori_loop(0, 16, rbody, 0)
                    return 0
                lax.fori_loop(0, _R // 16, lb_body, 0)

            for cc in range(F // 16):
                sl = pl.ds(cc * 16, 16)

                def wbody(r, _, sl=sl):
                    p_v[r, sl] = t_v[r, sl] + p_v[r, sl]
                    return 0
                lax.fori_loop(0, _R, wbody, 0)
            pltpu.sync_copy(p_v, out_hbm.at[pl.ds(lo, _R)])

    return seg_kernel


_seg170 = _make_segsum(E + N, True)
_seg160 = _make_segsum(E, False)


def _sc_segsum_agg(g, src, dst, off):
    gp = jnp.concatenate([g, jnp.zeros((_NPAD + 16 - N, F), jnp.float32)])
    offp = jnp.concatenate([off, jnp.zeros(_NPAD - N, jnp.int32)])
    nch = E // 16
    hit = jnp.zeros((_NRANGE * nch,), jnp.int32).at[
        (dst // _R) * nch + jnp.arange(E) // 16].set(1)
    ones = jnp.ones((E,), jnp.float32)
    lsc = jnp.zeros((_NPAD,), jnp.float32)
    out = _seg160(gp, src, dst, ones, hit, offp, lsc)
    return out[:N]


def _sc_segsum_conv(g, src, dst, scale, off, loop_scale):
    gp = jnp.concatenate([g, jnp.zeros((_NPAD + 16 - N, F), jnp.float32)])
    offp = jnp.concatenate([off, jnp.zeros(_NPAD - N, jnp.int32)])
    lscp = jnp.concatenate([loop_scale, jnp.zeros(_NPAD - N, jnp.float32)])
    nch = E // 16
    hit = jnp.zeros((_NRANGE * nch,), jnp.int32).at[
        (dst // _R) * nch + jnp.arange(E) // 16].set(1)
    out = _seg170(gp, src, dst, scale, hit, offp, lscp)
    return out[:N]


def kernel(x, edge_index, W1, b1, W2, b2, Wsl, Wsa, bs, Wsub, bsub):
    src, dst = edge_index[0], edge_index[1]
    loop = jnp.arange(N)
    s_all = jnp.concatenate([src, loop])
    d_all = jnp.concatenate([dst, loop])

    deg = jnp.zeros(N, jnp.float32).at[d_all].add(1.0)
    dinv = jnp.where(deg > 0, deg ** -0.5, 0.0)

    cnt = deg.astype(jnp.int32) - 1
    off1 = jnp.cumsum(cnt + 1) - (cnt + 1)
    off2 = jnp.cumsum(cnt) - cnt
    norm = dinv[s_all] * dinv[d_all]

    h1 = _mm(x, W1, 1000)
    out1 = _sc_segsum_conv(h1, src, dst, norm[:E], off1, dinv * dinv)
    x1 = jax.nn.relu(out1 + b1)

    agg = _sc_segsum_agg(x1, src, dst, off2)
    score = jnp.tanh((x1 @ Wsl + agg @ Wsa + bs).reshape(-1))

    scores_pad = jnp.concatenate([score, jnp.full((PN - N,), -jnp.inf, jnp.float32)])
    rank = _rank(scores_pad)
    perm_full = jnp.zeros(N, jnp.int32).at[jnp.clip(rank, 0, N - 1)].set(loop.astype(jnp.int32))
    perm = perm_full[:K]
    topv = score[perm]

    h2 = _mm(x1, W2, 1000)
    out2 = _sc_segsum_conv(h2, src, dst, norm[:E], off1, dinv * dinv)
    xc = jax.nn.relu(out2 + b2)

    keep = rank < K
    rs = rank[src]
    rd = rank[dst]
    valid = (rs < K) & (rd < K)
    s2 = jnp.where(valid, rs, 0)
    d2 = jnp.where(valid, rd, 0)
    vw = valid.astype(jnp.float32)

    x_pool = x1[perm] * topv[:, None]
    emb1 = jnp.concatenate([jnp.max(x_pool, 0, keepdims=True),
                            jnp.mean(x_pool, 0, keepdims=True)], axis=1)

    loop_k = jnp.arange(K)
    s2a = jnp.concatenate([s2, loop_k])
    d2a = jnp.concatenate([d2, loop_k])
    w2a = jnp.concatenate([vw, jnp.ones(K, jnp.float32)])
    deg2 = jnp.zeros(K, jnp.float32).at[d2a].add(w2a)
    dinv2 = jnp.where(deg2 > 0, deg2 ** -0.5, 0.0)
    norm2 = dinv2[s2a] * dinv2[d2a] * w2a
    hs = _mm(x_pool, Wsub, 1000)
    msgs = hs[s2a] * norm2[:, None]
    outs = jnp.zeros((K, F), jnp.float32).at[d2a].add(msgs)
    x_sub = jax.nn.relu(outs + bsub)
    emb2 = jnp.concatenate([jnp.max(x_sub, 0, keepdims=True),
                            jnp.mean(x_sub, 0, keepdims=True)], axis=1)

    pooled_edge_index = jnp.stack([s2, d2])
    batch = jnp.zeros((K,), jnp.int32)
    return (xc, emb1 + emb2, pooled_edge_index, perm, batch)


# speedup vs baseline: 1.0159x; 1.0159x over previous
import functools

import jax
import jax.numpy as jnp
from jax import lax
from jax.experimental import pallas as pl
from jax.experimental.pallas import tpu as pltpu
from jax.experimental.pallas import tpu_sc as plsc

N = 10000
E = 160000
F = 256
K = 5000

PN = 10240


def _mm_kernel(a_ref, b_ref, o_ref):
    o_ref[...] = jnp.dot(a_ref[...], b_ref[...], preferred_element_type=jnp.float32)


def _mm(a, b, bm):
    m, k = a.shape
    _, n = b.shape
    return pl.pallas_call(
        _mm_kernel,
        grid=(m // bm,),
        in_specs=[pl.BlockSpec((bm, k), lambda i: (i, 0)),
                  pl.BlockSpec((k, n), lambda i: (0, 0))],
        out_specs=pl.BlockSpec((bm, n), lambda i: (i, 0)),
        out_shape=jax.ShapeDtypeStruct((m, n), jnp.float32),
    )(a, b)


def _rank_kernel(si_ref, sall_ref, rank_ref):
    ib = pl.program_id(0)
    si = si_ref[...]
    gi = ib * 1024 + lax.broadcasted_iota(jnp.int32, (1024, 1), 0)
    acc = jnp.zeros((1024, 128), jnp.int32)
    for jc in range(PN // 128):
        sj = sall_ref[jc:jc + 1, :]
        gj = jc * 128 + lax.broadcasted_iota(jnp.int32, (1, 128), 1)
        gt = (sj > si)
        eq = (sj == si) & (gj < gi)
        acc = acc + (gt | eq).astype(jnp.int32)
    rank_ref[...] = jnp.sum(acc, axis=1, keepdims=True)


def _rank(scores_pad):
    s_col = scores_pad.reshape(PN, 1)
    s2d = scores_pad.reshape(PN // 128, 128)
    return pl.pallas_call(
        _rank_kernel,
        grid=(PN // 1024,),
        in_specs=[pl.BlockSpec((1024, 1), lambda i: (i, 0)),
                  pl.BlockSpec((PN // 128, 128), lambda i: (0, 0))],
        out_specs=pl.BlockSpec((1024, 1), lambda i: (i, 0)),
        out_shape=jax.ShapeDtypeStruct((PN, 1), jnp.int32),
    )(s_col, s2d).reshape(PN)[:N]


def _shard_bounds(u):
    sh = -(-(u // 16) // 16) * 16
    bounds = []
    p = 0
    t = 16
    while t > 2 and p + sh < u:
        bounds.append(p + sh)
        p += sh
        t -= 1
    rem = u - p
    half = -(-(rem + 1) // 2 // 16) * 16
    if rem > half:
        bounds.append(p + half)
    return bounds


def _exact_segsum(msg, dd, u, n_out, width):
    return jnp.zeros((n_out, width), jnp.float32).at[dd].add(msg)


_R = 160
_NRANGE = 64
_NPAD = _R * _NRANGE
_CH = 2000
_FL = 64


def _make_segsum(u, has_loop):
    bounds = _shard_bounds(u)
    nblk = E // _CH
    mesh = plsc.VectorSubcoreMesh(core_axis_name="c", subcore_axis_name="s")

    def vscalar(ref, idx):
        return ref[pl.ds(idx, 16)][0]

    @functools.partial(
        pl.kernel, mesh=mesh,
        out_type=jax.ShapeDtypeStruct((_NPAD, F), jnp.float32),
        scratch_types=[
            pltpu.VMEM((_R, F), jnp.float32),
            pltpu.VMEM((_R, F), jnp.float32),
            pltpu.VMEM((_FL, F), jnp.float32),
            pltpu.VMEM((_CH + 16,), jnp.int32),
            pltpu.VMEM((_CH + 16,), jnp.int32),
            pltpu.VMEM((_CH + 16,), jnp.float32),
            pltpu.VMEM((_FL + 32,), jnp.int32),
            pltpu.VMEM((E // 16 + 16,), jnp.int32),
            pltpu.VMEM((_R + 16,), jnp.int32),
            pltpu.VMEM((_R + 16,), jnp.float32),
            pltpu.SMEM((_FL + 16,), jnp.int32),
            pltpu.SMEM((_FL + 16,), jnp.float32),
            pltpu.SMEM((1,), jnp.int32),
            pltpu.SMEM((_R,), jnp.int32),
            pltpu.SMEM((_R,), jnp.float32),
            pltpu.SMEM((_R,), jnp.int32),
            pltpu.SMEM((_R,), jnp.int32),
            pltpu.SemaphoreType.DMA,
        ],
    )
    def seg_kernel(g_hbm, src_hbm, dst_hbm, sc_hbm, hit_hbm, off_hbm,
                   lsc_hbm, out_hbm,
                   t_v, p_v, rows_v, dblk_v, sblk_v, scblk_v,
                   gidx_v, hit_v, offst_v, lscst_v,
                   pd_s, pc_s, np_s, off_s, lsc_s, cnt_s, lastp_s, sem):
        wid = lax.axis_index("s") * 2 + lax.axis_index("c")
        zf16 = jnp.zeros((16,), jnp.float32)

        def piece_of(pos):
            pid = jnp.int32(0)
            for b in bounds:
                pid = pid + (pos >= b).astype(jnp.int32)
            return pid

        def accum(r, ld, scv):
            pos = off_s[ld] + cnt_s[ld]
            cnt_s[ld] = cnt_s[ld] + 1
            piece = piece_of(pos)
            lp = lastp_s[ld]
            lastp_s[ld] = piece

            @pl.when(piece > lp)
            def _():
                for cc in range(F // 16):
                    sl = pl.ds(cc * 16, 16)
                    t_v[ld, sl] = t_v[ld, sl] + p_v[ld, sl]
                    p_v[ld, sl] = zf16

            for cc in range(F // 16):
                sl = pl.ds(cc * 16, 16)
                p_v[ld, sl] = p_v[ld, sl] + rows_v[r, sl] * scv

        def gather_pending():
            pltpu.async_copy(g_hbm.at[gidx_v.at[pl.ds(0, _FL)]], rows_v,
                             sem).wait()

        for rng_i in range(2):
            rid = wid * 2 + rng_i
            lo = rid * _R

            pltpu.sync_copy(hit_hbm.at[pl.ds(rid * (E // 16), E // 16)],
                            hit_v.at[pl.ds(0, E // 16)])
            pltpu.sync_copy(off_hbm.at[pl.ds(lo, _R)],
                            offst_v.at[pl.ds(0, _R)])
            if has_loop:
                pltpu.sync_copy(lsc_hbm.at[pl.ds(lo, _R)],
                                lscst_v.at[pl.ds(0, _R)])

            def ibody(r, _):
                off_s[r] = vscalar(offst_v, r)
                if has_loop:
                    lsc_s[r] = vscalar(lscst_v, r)
                cnt_s[r] = 0
                lastp_s[r] = -1
                return 0
            lax.fori_loop(0, _R, ibody, 0)
            np_s[0] = 0

            for cc in range(F // 16):
                sl = pl.ds(cc * 16, 16)

                def zbody(r, _, sl=sl):
                    t_v[r, sl] = zf16
                    p_v[r, sl] = zf16
                    return 0
                lax.fori_loop(0, _R, zbody, 0)

            def blk_body(blk, _):
                eb = blk * _CH
                pltpu.sync_copy(dst_hbm.at[pl.ds(eb, _CH)],
                                dblk_v.at[pl.ds(0, _CH)])
                pltpu.sync_copy(src_hbm.at[pl.ds(eb, _CH)],
                                sblk_v.at[pl.ds(0, _CH)])
                pltpu.sync_copy(sc_hbm.at[pl.ds(eb, _CH)],
                                scblk_v.at[pl.ds(0, _CH)])

                def ch_body(ci, _):
                    base = ci * 16
                    c = vscalar(hit_v, blk * (_CH // 16) + ci)

                    @pl.when(c > 0)
                    def _():
                        def sift(r, _):
                            ld = vscalar(dblk_v, base + r) - lo

                            @pl.when((ld >= 0) & (ld < _R))
                            def _():
                                np0 = np_s[0]
                                pd_s[np0] = ld
                                pc_s[np0] = vscalar(scblk_v, base + r)
                                gidx_v[pl.ds(np0, 16)] = jnp.full(
                                    (16,), vscalar(sblk_v, base + r),
                                    jnp.int32)
                                np_s[0] = np0 + 1
                            return 0
                        lax.fori_loop(0, 16, sift, 0)

                    @pl.when(np_s[0] >= _FL)
                    def _():
                        gather_pending()

                        def fbody(r, _):
                            accum(r, pd_s[r], pc_s[r])
                            return 0
                        lax.fori_loop(0, _FL, fbody, 0)
                        rem = np_s[0] - _FL

                        def mbody(q, _):
                            @pl.when(q < rem)
                            def _():
                                pd_s[q] = pd_s[_FL + q]
                                pc_s[q] = pc_s[_FL + q]
                                gidx_v[pl.ds(q, 16)] = jnp.full(
                                    (16,), vscalar(gidx_v, _FL + q),
                                    jnp.int32)
                            return 0
                        lax.fori_loop(0, 16, mbody, 0)
                        np_s[0] = rem
                    return 0
                lax.fori_loop(0, _CH // 16, ch_body, 0)
                return 0
            lax.fori_loop(0, nblk, blk_body, 0)

            nleft = np_s[0]

            @pl.when(nleft > 0)
            def _():
                def padb(r, _):
                    @pl.when(r >= nleft)
                    def _():
                        gidx_v[pl.ds(r, 16)] = jnp.zeros((16,), jnp.int32)
                    return 0
                lax.fori_loop(0, _FL, padb, 0)
                gather_pending()

                def fbody(r, _):
                    @pl.when(r < nleft)
                    def _():
                        accum(r, pd_s[r], pc_s[r])
                    return 0
                lax.fori_loop(0, _FL, fbody, 0)

            if has_loop:
                def lb_body(b, _):
                    rb = b * 16
                    pltpu.async_copy(g_hbm.at[pl.ds(lo + rb, 16)],
                                     rows_v.at[pl.ds(0, 16)], sem).wait()

                    def rbody(r, _):
                        accum(r, rb + r, lsc_s[rb + r])
                        return 0
                    lax.fori_loop(0, 16, rbody, 0)
                    return 0
                lax.fori_loop(0, _R // 16, lb_body, 0)

            for cc in range(F // 16):
                sl = pl.ds(cc * 16, 16)

                def wbody(r, _, sl=sl):
                    p_v[r, sl] = t_v[r, sl] + p_v[r, sl]
                    return 0
                lax.fori_loop(0, _R, wbody, 0)
            pltpu.sync_copy(p_v, out_hbm.at[pl.ds(lo, _R)])

    return seg_kernel


_seg170 = _make_segsum(E + N, True)
_seg160 = _make_segsum(E, False)


def _sc_segsum_agg(g, src, dst, off):
    gp = jnp.concatenate([g, jnp.zeros((_NPAD + 16 - N, F), jnp.float32)])
    offp = jnp.concatenate([off, jnp.zeros(_NPAD - N, jnp.int32)])
    nch = E // 16
    hit = jnp.zeros((_NRANGE * nch,), jnp.int32).at[
        (dst // _R) * nch + jnp.arange(E) // 16].set(1)
    ones = jnp.ones((E,), jnp.float32)
    lsc = jnp.zeros((_NPAD,), jnp.float32)
    out = _seg160(gp, src, dst, ones, hit, offp, lsc)
    return out[:N]


def _sc_segsum_conv(g, src, dst, scale, off, loop_scale):
    gp = jnp.concatenate([g, jnp.zeros((_NPAD + 16 - N, F), jnp.float32)])
    offp = jnp.concatenate([off, jnp.zeros(_NPAD - N, jnp.int32)])
    lscp = jnp.concatenate([loop_scale, jnp.zeros(_NPAD - N, jnp.float32)])
    nch = E // 16
    hit = jnp.zeros((_NRANGE * nch,), jnp.int32).at[
        (dst // _R) * nch + jnp.arange(E) // 16].set(1)
    out = _seg170(gp, src, dst, scale, hit, offp, lscp)
    return out[:N]


def kernel(x, edge_index, W1, b1, W2, b2, Wsl, Wsa, bs, Wsub, bsub):
    src, dst = edge_index[0], edge_index[1]
    loop = jnp.arange(N)
    s_all = jnp.concatenate([src, loop])
    d_all = jnp.concatenate([dst, loop])

    deg = jnp.zeros(N, jnp.float32).at[d_all].add(1.0)
    dinv = jnp.where(deg > 0, deg ** -0.5, 0.0)

    cnt = deg.astype(jnp.int32) - 1
    off1 = jnp.cumsum(cnt + 1) - (cnt + 1)
    off2 = jnp.cumsum(cnt) - cnt
    norm = dinv[s_all] * dinv[d_all]

    h1 = _mm(x, W1, 1000)
    out1 = _sc_segsum_conv(h1, src, dst, norm[:E], off1, dinv * dinv)
    x1 = jax.nn.relu(out1 + b1)

    agg = _sc_segsum_agg(x1, src, dst, off2)
    score = jnp.tanh((x1 @ Wsl + agg @ Wsa + bs).reshape(-1))

    scores_pad = jnp.concatenate([score, jnp.full((PN - N,), -jnp.inf, jnp.float32)])
    rank = _rank(scores_pad)
    perm_full = jnp.zeros(N, jnp.int32).at[jnp.clip(rank, 0, N - 1)].set(loop.astype(jnp.int32))
    perm = perm_full[:K]
    topv = score[perm]

    h2 = _mm(x1, W2, 1000)
    out2 = _sc_segsum_conv(h2, src, dst, norm[:E], off1, dinv * dinv)
    xc = jax.nn.relu(out2 + b2)

    keep = rank < K
    rs = rank[src]
    rd = rank[dst]
    valid = (rs < K) & (rd < K)
    s2 = jnp.where(valid, rs, 0)
    d2 = jnp.where(valid, rd, 0)
    vw = valid.astype(jnp.float32)

    x_pool = x1[perm] * topv[:, None]
    emb1 = jnp.concatenate([jnp.max(x_pool, 0, keepdims=True),
                            jnp.mean(x_pool, 0, keepdims=True)], axis=1)

    loop_k = jnp.arange(K)
    s2a = jnp.concatenate([s2, loop_k])
    d2a = jnp.concatenate([d2, loop_k])
    w2a = jnp.concatenate([vw, jnp.ones(K, jnp.float32)])
    deg2 = jnp.zeros(K, jnp.float32).at[d2a].add(w2a)
    dinv2 = jnp.where(deg2 > 0, deg2 ** -0.5, 0.0)
    norm2 = dinv2[s2a] * dinv2[d2a] * w2a
    hs = _mm(x_pool, Wsub, 1000)
    msgs = hs[s2a] * norm2[:, None]
    outs = jnp.zeros((K, F), jnp.float32).at[d2a].add(msgs)
    x_sub = jax.nn.relu(outs + bsub)
    emb2 = jnp.concatenate([jnp.max(x_sub, 0, keepdims=True),
                            jnp.mean(x_sub, 0, keepdims=True)], axis=1)

    pooled_edge_index = jnp.stack([s2, d2])
    batch = jnp.zeros((K,), jnp.int32)
    return (xc, emb1 + emb2, pooled_edge_index, perm, batch)


# speedup vs baseline: 1.1792x; 1.1607x over previous
import functools

import jax
import jax.numpy as jnp
from jax import lax
from jax.experimental import pallas as pl
from jax.experimental.pallas import tpu as pltpu
from jax.experimental.pallas import tpu_sc as plsc

N = 10000
E = 160000
F = 256
K = 5000

PN = 10240


def _mm_kernel(a_ref, b_ref, o_ref):
    o_ref[...] = jnp.dot(a_ref[...], b_ref[...], preferred_element_type=jnp.float32)


def _mm(a, b, bm):
    m, k = a.shape
    _, n = b.shape
    return pl.pallas_call(
        _mm_kernel,
        grid=(m // bm,),
        in_specs=[pl.BlockSpec((bm, k), lambda i: (i, 0)),
                  pl.BlockSpec((k, n), lambda i: (0, 0))],
        out_specs=pl.BlockSpec((bm, n), lambda i: (i, 0)),
        out_shape=jax.ShapeDtypeStruct((m, n), jnp.float32),
    )(a, b)


def _rank_kernel(si_ref, sall_ref, rank_ref):
    ib = pl.program_id(0)
    si = si_ref[...]
    gi = ib * 1024 + lax.broadcasted_iota(jnp.int32, (1024, 1), 0)
    acc = jnp.zeros((1024, 128), jnp.int32)
    for jc in range(PN // 128):
        sj = sall_ref[jc:jc + 1, :]
        gj = jc * 128 + lax.broadcasted_iota(jnp.int32, (1, 128), 1)
        gt = (sj > si)
        eq = (sj == si) & (gj < gi)
        acc = acc + (gt | eq).astype(jnp.int32)
    rank_ref[...] = jnp.sum(acc, axis=1, keepdims=True)


def _rank(scores_pad):
    s_col = scores_pad.reshape(PN, 1)
    s2d = scores_pad.reshape(PN // 128, 128)
    return pl.pallas_call(
        _rank_kernel,
        grid=(PN // 1024,),
        in_specs=[pl.BlockSpec((1024, 1), lambda i: (i, 0)),
                  pl.BlockSpec((PN // 128, 128), lambda i: (0, 0))],
        out_specs=pl.BlockSpec((1024, 1), lambda i: (i, 0)),
        out_shape=jax.ShapeDtypeStruct((PN, 1), jnp.int32),
    )(s_col, s2d).reshape(PN)[:N]


def _shard_bounds(u):
    sh = -(-(u // 16) // 16) * 16
    bounds = []
    p = 0
    t = 16
    while t > 2 and p + sh < u:
        bounds.append(p + sh)
        p += sh
        t -= 1
    rem = u - p
    half = -(-(rem + 1) // 2 // 16) * 16
    if rem > half:
        bounds.append(p + half)
    return bounds


def _exact_segsum(msg, dd, u, n_out, width):
    return jnp.zeros((n_out, width), jnp.float32).at[dd].add(msg)


_R = 160
_NRANGE = 64
_NPAD = _R * _NRANGE
_CH = 2000
_FL = 64


def _make_segsum(u, has_loop):
    bounds = _shard_bounds(u)
    mesh = plsc.VectorSubcoreMesh(core_axis_name="c", subcore_axis_name="s")

    def vscalar(ref, idx):
        return ref[pl.ds(idx, 16)][0]

    @functools.partial(
        pl.kernel, mesh=mesh,
        out_type=jax.ShapeDtypeStruct((_NPAD, F), jnp.float32),
        scratch_types=[
            pltpu.VMEM((_R + 1, F), jnp.float32),
            pltpu.VMEM((_R + 1, F), jnp.float32),
            pltpu.VMEM((_FL, F), jnp.float32),
            pltpu.VMEM((_FL,), jnp.int32),
            pltpu.VMEM((_FL + 16,), jnp.int32),
            pltpu.VMEM((_FL + 16,), jnp.float32),
            pltpu.VMEM((_NRANGE + 16,), jnp.int32),
            pltpu.VMEM((_NRANGE + 16,), jnp.int32),
            pltpu.VMEM((_R + 16,), jnp.int32),
            pltpu.VMEM((_R + 16,), jnp.float32),
            pltpu.SMEM((_R + 1,), jnp.int32),
            pltpu.SMEM((_R + 1,), jnp.float32),
            pltpu.SMEM((_R + 1,), jnp.int32),
            pltpu.SMEM((_R + 1,), jnp.int32),
            pltpu.SemaphoreType.DMA,
        ],
    )
    def seg_kernel(g_hbm, srcg_hbm, dstg_hbm, scg_hbm, st_hbm, nb_hbm,
                   off_hbm, lsc_hbm, out_hbm,
                   t_v, p_v, rows_v, gidx_v, pd_v, pc_v, st_v, nb_v,
                   offst_v, lscst_v, off_s, lsc_s, cnt_s, lastp_s, sem):
        wid = lax.axis_index("s") * 2 + lax.axis_index("c")
        zf16 = jnp.zeros((16,), jnp.float32)

        def piece_of(pos):
            pid = jnp.int32(0)
            for b in bounds:
                pid = pid + (pos >= b).astype(jnp.int32)
            return pid

        def accum(r, ld, scv):
            pos = off_s[ld] + cnt_s[ld]
            cnt_s[ld] = cnt_s[ld] + 1
            piece = piece_of(pos)
            lp = lastp_s[ld]
            lastp_s[ld] = piece

            @pl.when(piece > lp)
            def _():
                for cc in range(F // 16):
                    sl = pl.ds(cc * 16, 16)
                    t_v[ld, sl] = t_v[ld, sl] + p_v[ld, sl]
                    p_v[ld, sl] = zf16

            for cc in range(F // 16):
                sl = pl.ds(cc * 16, 16)
                p_v[ld, sl] = p_v[ld, sl] + rows_v[r, sl] * scv

        pltpu.sync_copy(st_hbm, st_v.at[pl.ds(0, _NRANGE)])
        pltpu.sync_copy(nb_hbm, nb_v.at[pl.ds(0, _NRANGE)])

        for rng_i in range(2):
            rid = wid * 2 + rng_i
            lo = rid * _R

            pltpu.sync_copy(off_hbm.at[pl.ds(lo, _R)],
                            offst_v.at[pl.ds(0, _R)])
            if has_loop:
                pltpu.sync_copy(lsc_hbm.at[pl.ds(lo, _R)],
                                lscst_v.at[pl.ds(0, _R)])

            def ibody(r, _):
                off_s[r] = vscalar(offst_v, r)
                if has_loop:
                    lsc_s[r] = vscalar(lscst_v, r)
                cnt_s[r] = 0
                lastp_s[r] = -1
                return 0
            lax.fori_loop(0, _R, ibody, 0)

            for cc in range(F // 16):
                sl = pl.ds(cc * 16, 16)

                def zbody(r, _, sl=sl):
                    t_v[r, sl] = zf16
                    p_v[r, sl] = zf16
                    return 0
                lax.fori_loop(0, _R, zbody, 0)

            start = pl.multiple_of(vscalar(st_v, rid), _FL)
            nb = vscalar(nb_v, rid)

            def blk_body(b, _):
                eb = start + b * _FL
                pltpu.sync_copy(srcg_hbm.at[pl.ds(eb, _FL)], gidx_v)
                pltpu.sync_copy(dstg_hbm.at[pl.ds(eb, _FL)],
                                pd_v.at[pl.ds(0, _FL)])
                pltpu.sync_copy(scg_hbm.at[pl.ds(eb, _FL)],
                                pc_v.at[pl.ds(0, _FL)])
                pltpu.async_copy(g_hbm.at[gidx_v], rows_v, sem).wait()

                def fbody(r, _):
                    accum(r, vscalar(pd_v, r), vscalar(pc_v, r))
                    return 0
                lax.fori_loop(0, _FL, fbody, 0)
                return 0
            lax.fori_loop(0, nb, blk_body, 0)

            if has_loop:
                def lb_body(b, _):
                    rb = b * 16
                    pltpu.async_copy(g_hbm.at[pl.ds(lo + rb, 16)],
                                     rows_v.at[pl.ds(0, 16)], sem).wait()

                    def rbody(r, _):
                        accum(r, rb + r, lsc_s[rb + r])
                        return 0
                    lax.fori_loop(0, 16, rbody, 0)
                    return 0
                lax.fori_loop(0, _R // 16, lb_body, 0)

            for cc in range(F // 16):
                sl = pl.ds(cc * 16, 16)

                def wbody(r, _, sl=sl):
                    p_v[r, sl] = t_v[r, sl] + p_v[r, sl]
                    return 0
                lax.fori_loop(0, _R, wbody, 0)
            pltpu.sync_copy(p_v.at[pl.ds(0, _R)], out_hbm.at[pl.ds(lo, _R)])

    return seg_kernel


_seg170 = _make_segsum(E + N, True)
_seg160 = _make_segsum(E, False)

_EPAD = E + _NRANGE * _FL


def _group_edges(src, dst, scale):
    rid_e = dst // _R
    order = jnp.argsort(rid_e, stable=True)
    src_g = src[order]
    dst_g = dst[order] - rid_e[order] * _R
    sc_g = scale[order]
    rcnt = jnp.zeros(_NRANGE, jnp.int32).at[rid_e].add(1)
    nblk = -(-rcnt // _FL)
    starts = (jnp.cumsum(nblk) - nblk) * _FL
    epos = jnp.arange(E) - (jnp.cumsum(rcnt) - rcnt)[rid_e[order]]
    tgt = starts[rid_e[order]] + epos
    srcp = jnp.zeros(_EPAD, jnp.int32).at[tgt].set(src_g)
    dstp = jnp.full(_EPAD, _R, jnp.int32).at[tgt].set(dst_g)
    scp = jnp.zeros(_EPAD, jnp.float32).at[tgt].set(sc_g)
    return srcp, dstp, scp, starts.astype(jnp.int32), nblk.astype(jnp.int32)


def _sc_segsum_agg(g, src, dst, off):
    gp = jnp.concatenate([g, jnp.zeros((_NPAD + 16 - N, F), jnp.float32)])
    offp = jnp.concatenate([off, jnp.zeros(_NPAD - N, jnp.int32)])
    srcp, dstp, scp, starts, nblk = _group_edges(
        src, dst, jnp.ones((E,), jnp.float32))
    lsc = jnp.zeros((_NPAD,), jnp.float32)
    out = _seg160(gp, srcp, dstp, scp, starts, nblk, offp, lsc)
    return out[:N]


def _sc_segsum_conv(g, src, dst, scale, off, loop_scale):
    gp = jnp.concatenate([g, jnp.zeros((_NPAD + 16 - N, F), jnp.float32)])
    offp = jnp.concatenate([off, jnp.zeros(_NPAD - N, jnp.int32)])
    lscp = jnp.concatenate([loop_scale, jnp.zeros(_NPAD - N, jnp.float32)])
    srcp, dstp, scp, starts, nblk = _group_edges(src, dst, scale)
    out = _seg170(gp, srcp, dstp, scp, starts, nblk, offp, lscp)
    return out[:N]


def kernel(x, edge_index, W1, b1, W2, b2, Wsl, Wsa, bs, Wsub, bsub):
    src, dst = edge_index[0], edge_index[1]
    loop = jnp.arange(N)
    s_all = jnp.concatenate([src, loop])
    d_all = jnp.concatenate([dst, loop])

    deg = jnp.zeros(N, jnp.float32).at[d_all].add(1.0)
    dinv = jnp.where(deg > 0, deg ** -0.5, 0.0)

    cnt = deg.astype(jnp.int32) - 1
    off1 = jnp.cumsum(cnt + 1) - (cnt + 1)
    off2 = jnp.cumsum(cnt) - cnt
    norm = dinv[s_all] * dinv[d_all]

    h1 = _mm(x, W1, 1000)
    out1 = _sc_segsum_conv(h1, src, dst, norm[:E], off1, dinv * dinv)
    x1 = jax.nn.relu(out1 + b1)

    agg = _sc_segsum_agg(x1, src, dst, off2)
    score = jnp.tanh((x1 @ Wsl + agg @ Wsa + bs).reshape(-1))

    scores_pad = jnp.concatenate([score, jnp.full((PN - N,), -jnp.inf, jnp.float32)])
    rank = _rank(scores_pad)
    perm_full = jnp.zeros(N, jnp.int32).at[jnp.clip(rank, 0, N - 1)].set(loop.astype(jnp.int32))
    perm = perm_full[:K]
    topv = score[perm]

    h2 = _mm(x1, W2, 1000)
    out2 = _sc_segsum_conv(h2, src, dst, norm[:E], off1, dinv * dinv)
    xc = jax.nn.relu(out2 + b2)

    keep = rank < K
    rs = rank[src]
    rd = rank[dst]
    valid = (rs < K) & (rd < K)
    s2 = jnp.where(valid, rs, 0)
    d2 = jnp.where(valid, rd, 0)
    vw = valid.astype(jnp.float32)

    x_pool = x1[perm] * topv[:, None]
    emb1 = jnp.concatenate([jnp.max(x_pool, 0, keepdims=True),
                            jnp.mean(x_pool, 0, keepdims=True)], axis=1)

    loop_k = jnp.arange(K)
    s2a = jnp.concatenate([s2, loop_k])
    d2a = jnp.concatenate([d2, loop_k])
    w2a = jnp.concatenate([vw, jnp.ones(K, jnp.float32)])
    deg2 = jnp.zeros(K, jnp.float32).at[d2a].add(w2a)
    dinv2 = jnp.where(deg2 > 0, deg2 ** -0.5, 0.0)
    norm2 = dinv2[s2a] * dinv2[d2a] * w2a
    hs = _mm(x_pool, Wsub, 1000)
    msgs = hs[s2a] * norm2[:, None]
    outs = jnp.zeros((K, F), jnp.float32).at[d2a].add(msgs)
    x_sub = jax.nn.relu(outs + bsub)
    emb2 = jnp.concatenate([jnp.max(x_sub, 0, keepdims=True),
                            jnp.mean(x_sub, 0, keepdims=True)], axis=1)

    pooled_edge_index = jnp.stack([s2, d2])
    batch = jnp.zeros((K,), jnp.int32)
    return (xc, emb1 + emb2, pooled_edge_index, perm, batch)
